# Initial kernel scaffold; baseline (speedup 1.0000x reference)
import jax
import jax.numpy as jnp
from jax.experimental import pallas as pl


def kernel(x, edge_index, ntype, etype, Wk0, Wq0, Wv0, Wa0, Watt0, Wmsg0, pri0, skip0, lng0, lnb0, Wk1, Wq1, Wv1, Wa1, Watt1, Wmsg1, pri1, skip1, lng1, lnb1, agg_w, agg_g, agg_b):
    raise NotImplementedError("write your pallas kernel here")


# speedup vs baseline: 1.0006x; 1.0006x over previous
import jax
import jax.numpy as jnp
from jax.experimental import pallas as pl

N = 10000
E = 320000
D = 128
OUT = 128
H = 8
HD = 16
NT = 3
ET = 4
L = 2


def _layer_norm(x, g, b, eps=1e-5):
    mu = jnp.mean(x, axis=-1, keepdims=True)
    var = jnp.mean((x - mu) ** 2, axis=-1, keepdims=True)
    return (x - mu) / jnp.sqrt(var + eps) * g + b


def _typed_linear(x, types, W):
    allo = jnp.einsum('ni,tio->nto', x, W)
    return jnp.take_along_axis(allo, types[:, None, None], axis=1)[:, 0, :]


def _hgt_layer(x, src, dst, ntype, etype, Wk, Wq, Wv, Wa, Watt, Wmsg, pri, skip, lng, lnb):
    k = _typed_linear(x, ntype, Wk).reshape(N, H, HD)
    q = _typed_linear(x, ntype, Wq).reshape(N, H, HD)
    v = _typed_linear(x, ntype, Wv).reshape(N, H, HD)
    ke = k[src]
    qe = q[dst]
    ve = v[src]
    sqrt_d = float(HD) ** 0.5
    a_list = []
    m_list = []
    for h in range(H):
        kw = _typed_linear(ke[:, h, :], etype, Watt[h])
        a_list.append(jnp.sum(kw * qe[:, h, :], axis=-1) * pri[h][etype] / sqrt_d)
        m_list.append(_typed_linear(ve[:, h, :], etype, Wmsg[h]))
    A = jnp.stack(a_list, axis=1)
    M = jnp.stack(m_list, axis=1)
    amax = jax.ops.segment_max(A, dst, num_segments=N)
    amax = jnp.where(jnp.isfinite(amax), amax, 0.0)
    expA = jnp.exp(A - amax[dst])
    denom = jax.ops.segment_sum(expA, dst, num_segments=N)
    attn = expA / jnp.maximum(denom[dst], 1e-12)
    hmsg = jax.ops.segment_sum(M * attn[:, :, None], dst, num_segments=N).reshape(N, H * HD)
    hout = _typed_linear(hmsg, ntype, Wa)
    alpha = jax.nn.sigmoid(skip)[ntype][:, None]
    hout = hout * alpha + x * (1.0 - alpha)
    return _layer_norm(x + hout, lng, lnb)


def _final_ln_kernel(mixed_ref, g_ref, b_ref, o_ref):
    xb = mixed_ref[...]
    mu = jnp.mean(xb, axis=-1, keepdims=True)
    var = jnp.mean((xb - mu) ** 2, axis=-1, keepdims=True)
    o_ref[...] = (xb - mu) / jnp.sqrt(var + 1e-5) * g_ref[...] + b_ref[...]


def kernel(x, edge_index, ntype, etype, Wk0, Wq0, Wv0, Wa0, Watt0, Wmsg0, pri0, skip0, lng0, lnb0, Wk1, Wq1, Wv1, Wa1, Watt1, Wmsg1, pri1, skip1, lng1, lnb1, agg_w, agg_g, agg_b):
    src = edge_index[0]
    dst = edge_index[1]
    o0 = _hgt_layer(x, src, dst, ntype, etype, Wk0, Wq0, Wv0, Wa0, Watt0, Wmsg0, pri0, skip0, lng0, lnb0)
    o1 = _hgt_layer(o0, src, dst, ntype, etype, Wk1, Wq1, Wv1, Wa1, Watt1, Wmsg1, pri1, skip1, lng1, lnb1)
    w = jax.nn.softmax(agg_w)
    mixed = w[0] * o0 + w[1] * o1
    return pl.pallas_call(
        _final_ln_kernel,
        out_shape=jax.ShapeDtypeStruct((N, OUT), jnp.float32),
        grid=(10,),
        in_specs=[
            pl.BlockSpec((N // 10, OUT), lambda i: (i, 0)),
            pl.BlockSpec((OUT,), lambda i: (0,)),
            pl.BlockSpec((OUT,), lambda i: (0,)),
        ],
        out_specs=pl.BlockSpec((N // 10, OUT), lambda i: (i, 0)),
    )(mixed, agg_g, agg_b)


# speedup vs baseline: 13.1087x; 13.1008x over previous
import functools

import jax
import jax.numpy as jnp
from jax import lax
from jax.experimental import pallas as pl
from jax.experimental.pallas import tpu as pltpu
from jax.experimental.pallas import tpu_sc as plsc

N = 10000
E = 320000
D = 128
OUT = 128
H = 8
HD = 16
NT = 3
ET = 4

NTILES = 32
EPT = 10240
EPAD = EPT * NTILES
C = 256
NCHUNK = EPT // C
NGRP = C // 16
SUB = 128
NSUB = C // SUB
C2 = 64
NSUBC = C // C2
NGRP2 = C2 // 16
NP = 10240
ROWS_T = NP // 16
NZP = ROWS_T // C2
DREG = NP // 16
DROWS_T = DREG // 16
TROWS = NP + DREG


def _stage_a_body(x_ref, nt_ref, wk_ref, wq_ref, wv_ref, biga_ref, bigm_ref,
                  kt_ref, vt_ref, q_ref):
    xb = x_ref[...]
    nt = nt_ref[...]
    k = jnp.zeros_like(xb)
    q = jnp.zeros_like(xb)
    v = jnp.zeros_like(xb)
    for t in range(NT):
        m = (nt == float(t)).astype(jnp.float32)
        k = k + m * jnp.dot(xb, wk_ref[t], preferred_element_type=jnp.float32)
        q = q + m * jnp.dot(xb, wq_ref[t], preferred_element_type=jnp.float32)
        v = v + m * jnp.dot(xb, wv_ref[t], preferred_element_type=jnp.float32)
    kt_ref[...] = jnp.dot(k, biga_ref[...], preferred_element_type=jnp.float32)
    vt_ref[...] = jnp.dot(v, bigm_ref[...], preferred_element_type=jnp.float32)
    q_ref[...] = q


def _stage_a(x, nt2d, Wk, Wq, Wv, BigA, BigM):
    B = 1000
    g = N // B
    return pl.pallas_call(
        _stage_a_body,
        grid=(g,),
        in_specs=[
            pl.BlockSpec((B, D), lambda i: (i, 0)),
            pl.BlockSpec((B, 1), lambda i: (i, 0)),
            pl.BlockSpec((NT, D, OUT), lambda i: (0, 0, 0)),
            pl.BlockSpec((NT, D, OUT), lambda i: (0, 0, 0)),
            pl.BlockSpec((NT, D, OUT), lambda i: (0, 0, 0)),
            pl.BlockSpec((OUT, ET * OUT), lambda i: (0, 0)),
            pl.BlockSpec((OUT, ET * OUT), lambda i: (0, 0)),
        ],
        out_specs=[
            pl.BlockSpec((B, ET * OUT), lambda i: (i, 0)),
            pl.BlockSpec((B, ET * OUT), lambda i: (i, 0)),
            pl.BlockSpec((B, OUT), lambda i: (i, 0)),
        ],
        out_shape=[
            jax.ShapeDtypeStruct((N, ET * OUT), jnp.float32),
            jax.ShapeDtypeStruct((N, ET * OUT), jnp.float32),
            jax.ShapeDtypeStruct((N, OUT), jnp.float32),
        ],
    )(x, nt2d, Wk, Wq, Wv, BigA, BigM)


def _sc_pass1(kt, q, idx2d, dst2d):
    mesh = plsc.VectorSubcoreMesh(core_axis_name="c", subcore_axis_name="s")

    @functools.partial(
        pl.kernel,
        mesh=mesh,
        compiler_params=pltpu.CompilerParams(needs_layout_passes=False),
        out_type=[
            jax.ShapeDtypeStruct((NTILES, NCHUNK, H, C), jnp.float32),
            jax.ShapeDtypeStruct((NTILES, 16), jnp.float32),
        ],
        scratch_types=[
            pltpu.VMEM((C, OUT), jnp.float32),
            pltpu.VMEM((C, OUT), jnp.float32),
            pltpu.VMEM((H, C), jnp.float32),
            pltpu.VMEM((NSUB, SUB), jnp.int32),
            pltpu.VMEM((NSUB, SUB), jnp.int32),
            pltpu.VMEM((16,), jnp.float32),
            pltpu.SemaphoreType.DMA,
        ],
    )
    def k1(kt_hbm, q_hbm, idx_hbm, dst_hbm, a_hbm, tmax_hbm,
           kb, qb, ab, idx_v, dst_v, tm_v, sem):
        cid = lax.axis_index("c")
        sid = lax.axis_index("s")
        wid = cid * 16 + sid
        iota = lax.iota(jnp.int32, 16)

        def chunk_body(i, tmaxes):
            r0 = wid * (EPT // SUB) + i * NSUB
            pltpu.sync_copy(idx_hbm.at[pl.ds(r0, NSUB)], idx_v)
            pltpu.sync_copy(dst_hbm.at[pl.ds(r0, NSUB)], dst_v)
            for p in range(NSUB):
                pltpu.async_copy(kt_hbm.at[idx_v.at[p]],
                                 kb.at[pl.ds(p * SUB, SUB)], sem).wait()
                pltpu.async_copy(q_hbm.at[dst_v.at[p]],
                                 qb.at[pl.ds(p * SUB, SUB)], sem).wait()

            def grp_body(g, tm):
                rows = g * 16 + iota
                acc = [jnp.zeros((16,), jnp.float32) for _ in range(H)]
                for j in range(OUT):
                    jv = jnp.full((16,), j, jnp.int32)
                    kc = plsc.load_gather(kb, [rows, jv])
                    qc = plsc.load_gather(qb, [rows, jv])
                    acc[j // HD] = acc[j // HD] + kc * qc
                new_tm = []
                for h in range(H):
                    hv = jnp.full((16,), h, jnp.int32)
                    plsc.store_scatter(ab, [hv, rows], acc[h])
                    new_tm.append(jnp.maximum(tm[h], acc[h]))
                return tuple(new_tm)

            tmaxes = lax.fori_loop(0, NGRP, grp_body, tmaxes)
            pltpu.sync_copy(ab, a_hbm.at[wid, i])
            return tmaxes

        init = tuple(jnp.full((16,), -jnp.inf, jnp.float32) for _ in range(H))
        tmaxes = lax.fori_loop(0, NCHUNK, chunk_body, init)
        tm = tmaxes[0]
        for h in range(1, H):
            tm = jnp.maximum(tm, tmaxes[h])
        tm_v[...] = tm
        pltpu.sync_copy(tm_v, tmax_hbm.at[wid])

    return k1(kt, q, idx2d, dst2d)


def _sc_pass2(vt, a, idx2d, dst2d, dden2d, gmax16):
    mesh = plsc.VectorSubcoreMesh(core_axis_name="c", subcore_axis_name="s")

    @functools.partial(
        pl.kernel,
        mesh=mesh,
        compiler_params=pltpu.CompilerParams(needs_layout_passes=False),
        out_type=[
            jax.ShapeDtypeStruct((2, NP, OUT), jnp.float32),
            jax.ShapeDtypeStruct((2, DREG, OUT), jnp.float32),
        ],
        scratch_types=[
            pltpu.VMEM((C2, OUT), jnp.float32),
            pltpu.VMEM((C2, OUT), jnp.float32),
            pltpu.VMEM((C2, OUT), jnp.float32),
            pltpu.VMEM((H, C), jnp.float32),
            pltpu.VMEM((C2,), jnp.int32),
            pltpu.VMEM((1, C2), jnp.int32),
            pltpu.VMEM((1, C2), jnp.int32),
            pltpu.VMEM((16,), jnp.float32),
            pltpu.VMEM_SHARED((TROWS, OUT), jnp.float32),
            pltpu.SemaphoreType.DMA,
        ],
    )
    def k2(vt_hbm, a_hbm, idx_hbm, dst_hbm, dden_hbm, gmax_hbm, num_hbm, den_hbm,
           vb, mb, db, ab, idx_v, dst_v, dden_v, gv, t_sh, sem):
        cid = lax.axis_index("c")
        sid = lax.axis_index("s")
        wid = cid * 16 + sid
        iota = lax.iota(jnp.int32, 16)
        zero16 = jnp.zeros((16,), jnp.float32)
        zero16i = jnp.zeros((16,), jnp.int32)

        def zmb(i, _):
            r = i // 8
            cidx = (i % 8) * 16
            mb[r, pl.ds(cidx, 16)] = zero16
            db[r, pl.ds(cidx, 16)] = zero16
            return 0
        lax.fori_loop(0, C2 * 8, zmb, 0)

        base = sid * ROWS_T
        for p in range(NZP):
            pltpu.sync_copy(mb, t_sh.at[pl.ds(base + p * C2, C2)])
        dbase = NP + sid * DROWS_T
        pltpu.sync_copy(mb.at[pl.ds(0, DROWS_T)], t_sh.at[pl.ds(dbase, DROWS_T)])
        plsc.subcore_barrier()

        pltpu.sync_copy(gmax_hbm, gv)
        g16 = gv[...]

        def chunk_body(io, _):
            pltpu.sync_copy(a_hbm.at[wid, io], ab)

            def sub_body(isb, _):
                r0 = wid * (NCHUNK * NSUBC) + io * NSUBC + isb
                pltpu.sync_copy(idx_hbm.at[r0], idx_v)
                pltpu.sync_copy(dst_hbm.at[pl.ds(r0, 1)], dst_v)
                pltpu.sync_copy(dden_hbm.at[pl.ds(r0, 1)], dden_v)
                pltpu.async_copy(vt_hbm.at[idx_v], vb, sem).wait()
                acol0 = isb * C2

                def grp_body(g, _):
                    rows = g * 16 + iota
                    dv = plsc.load_gather(dst_v, [zero16i, rows])
                    bcol = (dv & 15) * 8
                    ex = []
                    for h in range(H):
                        hv = jnp.full((16,), h, jnp.int32)
                        av = plsc.load_gather(ab, [hv, acol0 + rows])
                        e = jnp.exp(av - g16)
                        ex.append(e)
                        plsc.store_scatter(db, [rows, bcol + h], e)
                    for j in range(OUT):
                        jv = jnp.full((16,), j, jnp.int32)
                        vc = plsc.load_gather(vb, [rows, jv])
                        plsc.store_scatter(mb, [rows, jv], vc * ex[j // HD])
                    return 0

                lax.fori_loop(0, NGRP2, grp_body, 0)
                pltpu.sync_copy(mb, t_sh.at[dst_v.at[0]], add=True)
                pltpu.sync_copy(db, t_sh.at[dden_v.at[0]], add=True)

                def zgrp_body(g, _):
                    rows = g * 16 + iota
                    dv = plsc.load_gather(dst_v, [zero16i, rows])
                    bcol = (dv & 15) * 8
                    for h in range(H):
                        plsc.store_scatter(db, [rows, bcol + h], zero16)
                    return 0

                lax.fori_loop(0, NGRP2, zgrp_body, 0)
                return 0

            lax.fori_loop(0, NSUBC, sub_body, 0)
            return 0

        lax.fori_loop(0, NCHUNK, chunk_body, 0)
        plsc.subcore_barrier()

        for p in range(NZP):
            st = base + p * C2
            pltpu.sync_copy(t_sh.at[pl.ds(st, C2)], mb)
            pltpu.sync_copy(mb, num_hbm.at[cid].at[pl.ds(st, C2)])
        pltpu.sync_copy(t_sh.at[pl.ds(dbase, DROWS_T)], mb.at[pl.ds(0, DROWS_T)])
        pltpu.sync_copy(mb.at[pl.ds(0, DROWS_T)],
                        den_hbm.at[cid].at[pl.ds(sid * DROWS_T, DROWS_T)])

    return k2(vt, a, idx2d, dst2d, dden2d, gmax16)


def _stage_c_body(is_final, num0_ref, num1_ref, den0_ref, den1_ref, x_ref,
                  nt_ref, wa_ref, sk_ref, ln_ref, o0_ref, agg_ref, o_ref):
    den = den0_ref[0] + den1_ref[0]
    deninv = 1.0 / jnp.maximum(den, 1e-30)
    hexp = (lax.broadcasted_iota(jnp.int32, (H, OUT), 1) // HD ==
            lax.broadcasted_iota(jnp.int32, (H, OUT), 0)).astype(jnp.float32)
    denfull = jnp.dot(deninv, hexp, preferred_element_type=jnp.float32)
    hmsg = (num0_ref[0] + num1_ref[0]) * denfull
    nt = nt_ref[...]
    xb = x_ref[...]
    hout = jnp.zeros_like(xb)
    alpha = jnp.zeros_like(nt)
    for t in range(NT):
        m = (nt == float(t)).astype(jnp.float32)
        hout = hout + m * jnp.dot(hmsg, wa_ref[t], preferred_element_type=jnp.float32)
        alpha = alpha + m * sk_ref[0, t]
    y = xb + hout * alpha + xb * (1.0 - alpha)
    mu = jnp.mean(y, axis=-1, keepdims=True)
    var = jnp.mean((y - mu) ** 2, axis=-1, keepdims=True)
    o = (y - mu) / jnp.sqrt(var + 1e-5) * ln_ref[0, :] + ln_ref[1, :]
    if is_final:
        mixed = agg_ref[0, 0] * o0_ref[...] + agg_ref[0, 1] * o
        mu2 = jnp.mean(mixed, axis=-1, keepdims=True)
        var2 = jnp.mean((mixed - mu2) ** 2, axis=-1, keepdims=True)
        o = (mixed - mu2) / jnp.sqrt(var2 + 1e-5) * ln_ref[2, :] + ln_ref[3, :]
    o_ref[...] = o


def _stage_c(is_final, num, den, x, nt2d, Wa, skpad, lnstack, o0, aggpad):
    B = 1000
    g = N // B
    return pl.pallas_call(
        functools.partial(_stage_c_body, is_final),
        grid=(g,),
        in_specs=[
            pl.BlockSpec((1, B, OUT), lambda i: (0, i, 0)),
            pl.BlockSpec((1, B, OUT), lambda i: (1, i, 0)),
            pl.BlockSpec((1, B, H), lambda i: (0, i, 0)),
            pl.BlockSpec((1, B, H), lambda i: (1, i, 0)),
            pl.BlockSpec((B, D), lambda i: (i, 0)),
            pl.BlockSpec((B, 1), lambda i: (i, 0)),
            pl.BlockSpec((NT, OUT, OUT), lambda i: (0, 0, 0)),
            pl.BlockSpec((1, NT), lambda i: (0, 0)),
            pl.BlockSpec((4, OUT), lambda i: (0, 0)),
            pl.BlockSpec((B, OUT), lambda i: (i, 0)),
            pl.BlockSpec((1, 2), lambda i: (0, 0)),
        ],
        out_specs=pl.BlockSpec((B, OUT), lambda i: (i, 0)),
        out_shape=jax.ShapeDtypeStruct((N, OUT), jnp.float32),
    )(num, num, den, den, x, nt2d, Wa, skpad, lnstack, o0, aggpad)


def _big_block_diag(Wper, scale):
    t = Wper * scale[:, :, None, None]
    eye = jnp.eye(H, dtype=t.dtype)
    big = jnp.einsum('heij,hg->hiegj', t, eye)
    return big.reshape(H * HD, ET * H * HD)


def _hgt_layer_sc(x, nt2d, idx2d, dst2d, Wk, Wq, Wv, Wa, Watt, Wmsg, pri,
                  skip, lng, lnb, is_final, o0, aggpad, agg_g, agg_b):
    sqrt_d = float(HD) ** 0.5
    BigA = _big_block_diag(Watt, pri / sqrt_d)
    BigM = _big_block_diag(Wmsg, jnp.ones_like(pri))
    kt, vt, q = _stage_a(x, nt2d, Wk, Wq, Wv, BigA, BigM)
    kt = kt.reshape(N * ET, OUT)
    vt = vt.reshape(N * ET, OUT)
    a, tmax = _sc_pass1(kt, q, idx2d[0], dst2d[0])
    gmax16 = jnp.broadcast_to(jnp.max(tmax), (16,))
    num, denraw = _sc_pass2(vt, a, idx2d[1], dst2d[1], dst2d[2], gmax16)
    den = denraw.reshape(2, DREG, 16, H).reshape(2, NP, H)
    skpad = jnp.concatenate([jax.nn.sigmoid(skip)[None, :]], axis=0)
    if is_final:
        lnstack = jnp.stack([lng, lnb, agg_g, agg_b], axis=0)
    else:
        lnstack = jnp.stack([lng, lnb, jnp.zeros_like(lng), jnp.zeros_like(lnb)], axis=0)
    return _stage_c(is_final, num, den, x, nt2d, Wa, skpad, lnstack, o0, aggpad)


def kernel(x, edge_index, ntype, etype, Wk0, Wq0, Wv0, Wa0, Watt0, Wmsg0, pri0, skip0, lng0, lnb0, Wk1, Wq1, Wv1, Wa1, Watt1, Wmsg1, pri1, skip1, lng1, lnb1, agg_w, agg_g, agg_b):
    src = edge_index[0]
    dst = edge_index[1]
    idx = src * ET + etype
    pad = EPAD - E
    idxp = jnp.concatenate([idx, jnp.zeros((pad,), jnp.int32)])
    dstp = jnp.concatenate([dst, jnp.full((pad,), N, jnp.int32)])
    ddenp = NP + (dstp >> 4)
    idx2d = (idxp.reshape(EPAD // SUB, SUB), idxp.reshape(EPAD // C2, C2))
    dst2d = (dstp.reshape(EPAD // SUB, SUB), dstp.reshape(EPAD // C2, C2),
             ddenp.reshape(EPAD // C2, C2))
    nt2d = ntype.astype(jnp.float32).reshape(N, 1)
    w = jax.nn.softmax(agg_w)
    aggpad = w.reshape(1, 2)
    zed = jnp.zeros((N, OUT), jnp.float32)
    o0 = _hgt_layer_sc(x, nt2d, idx2d, dst2d, Wk0, Wq0, Wv0, Wa0, Watt0,
                       Wmsg0, pri0, skip0, lng0, lnb0, False, zed, aggpad,
                       agg_g, agg_b)
    out = _hgt_layer_sc(o0, nt2d, idx2d, dst2d, Wk1, Wq1, Wv1, Wa1, Watt1,
                        Wmsg1, pri1, skip1, lng1, lnb1, True, o0, aggpad,
                        agg_g, agg_b)
    return out


# speedup vs baseline: 15.3900x; 1.1740x over previous
import functools

import jax
import jax.numpy as jnp
from jax import lax
from jax.experimental import pallas as pl
from jax.experimental.pallas import tpu as pltpu
from jax.experimental.pallas import tpu_sc as plsc

N = 10000
E = 320000
D = 128
OUT = 128
H = 8
HD = 16
NT = 3
ET = 4

NTILES = 32
EPT = 10240
EPAD = EPT * NTILES
C = 256
NCHUNK = EPT // C
NGRP = C // 16
SUB = 128
NSUB = C // SUB
C2 = 64
NSUBC = C // C2
NGRP2 = C2 // 16
NP = 10240
ROWS_T = NP // 16
NZP = ROWS_T // C2
DREG = NP // 16
DROWS_T = DREG // 16
TROWS = NP + DREG


def _stage_a_body(x_ref, nt_ref, wk_ref, wq_ref, wv_ref, biga_ref, bigm_ref,
                  kt_ref, vt_ref, q_ref):
    xb = x_ref[...]
    nt = nt_ref[...]
    k = jnp.zeros_like(xb)
    q = jnp.zeros_like(xb)
    v = jnp.zeros_like(xb)
    for t in range(NT):
        m = (nt == float(t)).astype(jnp.float32)
        k = k + m * jnp.dot(xb, wk_ref[t], preferred_element_type=jnp.float32)
        q = q + m * jnp.dot(xb, wq_ref[t], preferred_element_type=jnp.float32)
        v = v + m * jnp.dot(xb, wv_ref[t], preferred_element_type=jnp.float32)
    kt_ref[...] = jnp.dot(k, biga_ref[...], preferred_element_type=jnp.float32)
    vt_ref[...] = jnp.dot(v, bigm_ref[...], preferred_element_type=jnp.float32)
    q_ref[...] = q


def _stage_a(x, nt2d, Wk, Wq, Wv, BigA, BigM):
    B = 1000
    g = N // B
    return pl.pallas_call(
        _stage_a_body,
        grid=(g,),
        in_specs=[
            pl.BlockSpec((B, D), lambda i: (i, 0)),
            pl.BlockSpec((B, 1), lambda i: (i, 0)),
            pl.BlockSpec((NT, D, OUT), lambda i: (0, 0, 0)),
            pl.BlockSpec((NT, D, OUT), lambda i: (0, 0, 0)),
            pl.BlockSpec((NT, D, OUT), lambda i: (0, 0, 0)),
            pl.BlockSpec((OUT, ET * OUT), lambda i: (0, 0)),
            pl.BlockSpec((OUT, ET * OUT), lambda i: (0, 0)),
        ],
        out_specs=[
            pl.BlockSpec((B, ET * OUT), lambda i: (i, 0)),
            pl.BlockSpec((B, ET * OUT), lambda i: (i, 0)),
            pl.BlockSpec((B, OUT), lambda i: (i, 0)),
        ],
        out_shape=[
            jax.ShapeDtypeStruct((N, ET * OUT), jnp.float32),
            jax.ShapeDtypeStruct((N, ET * OUT), jnp.float32),
            jax.ShapeDtypeStruct((N, OUT), jnp.float32),
        ],
    )(x, nt2d, Wk, Wq, Wv, BigA, BigM)


def _sc_pass1(kt, q, idx2d, dst2d):
    mesh = plsc.VectorSubcoreMesh(core_axis_name="c", subcore_axis_name="s")

    @functools.partial(
        pl.kernel,
        mesh=mesh,
        compiler_params=pltpu.CompilerParams(needs_layout_passes=False),
        out_type=[
            jax.ShapeDtypeStruct((NTILES, NCHUNK, H, C), jnp.float32),
            jax.ShapeDtypeStruct((NTILES, 16), jnp.float32),
        ],
        scratch_types=[
            pltpu.VMEM((C, OUT), jnp.float32),
            pltpu.VMEM((C, OUT), jnp.float32),
            pltpu.VMEM((H, C), jnp.float32),
            pltpu.VMEM((EPT // SUB, SUB), jnp.int32),
            pltpu.VMEM((EPT // SUB, SUB), jnp.int32),
            pltpu.VMEM((16,), jnp.float32),
            pltpu.SemaphoreType.DMA,
        ],
    )
    def k1(kt_hbm, q_hbm, idx_hbm, dst_hbm, a_hbm, tmax_hbm,
           kb, qb, ab, idx_v, dst_v, tm_v, sem):
        cid = lax.axis_index("c")
        sid = lax.axis_index("s")
        wid = cid * 16 + sid
        iota = lax.iota(jnp.int32, 16)

        pltpu.sync_copy(idx_hbm.at[pl.ds(wid * (EPT // SUB), EPT // SUB)], idx_v)
        pltpu.sync_copy(dst_hbm.at[pl.ds(wid * (EPT // SUB), EPT // SUB)], dst_v)

        def chunk_body(i, tmaxes):
            ds = []
            for p in range(NSUB):
                ds.append(pltpu.async_copy(kt_hbm.at[idx_v.at[i * NSUB + p]],
                                           kb.at[pl.ds(p * SUB, SUB)], sem))
                ds.append(pltpu.async_copy(q_hbm.at[dst_v.at[i * NSUB + p]],
                                           qb.at[pl.ds(p * SUB, SUB)], sem))
            for d in ds:
                d.wait()

            def grp_body(g, tm):
                rows = g * 16 + iota
                acc = [jnp.zeros((16,), jnp.float32) for _ in range(H)]
                for j in range(OUT):
                    jv = jnp.full((16,), j, jnp.int32)
                    kc = plsc.load_gather(kb, [rows, jv])
                    qc = plsc.load_gather(qb, [rows, jv])
                    acc[j // HD] = acc[j // HD] + kc * qc
                new_tm = []
                for h in range(H):
                    hv = jnp.full((16,), h, jnp.int32)
                    plsc.store_scatter(ab, [hv, rows], acc[h])
                    new_tm.append(jnp.maximum(tm[h], acc[h]))
                return tuple(new_tm)

            tmaxes = lax.fori_loop(0, NGRP, grp_body, tmaxes)
            pltpu.sync_copy(ab, a_hbm.at[wid, i])
            return tmaxes

        init = tuple(jnp.full((16,), -jnp.inf, jnp.float32) for _ in range(H))
        tmaxes = lax.fori_loop(0, NCHUNK, chunk_body, init)
        tm = tmaxes[0]
        for h in range(1, H):
            tm = jnp.maximum(tm, tmaxes[h])
        tm_v[...] = tm
        pltpu.sync_copy(tm_v, tmax_hbm.at[wid])

    return k1(kt, q, idx2d, dst2d)


def _sc_pass2(vt, a, idx2d, dst2d, dden2d, gmax16):
    mesh = plsc.VectorSubcoreMesh(core_axis_name="c", subcore_axis_name="s")

    @functools.partial(
        pl.kernel,
        mesh=mesh,
        compiler_params=pltpu.CompilerParams(needs_layout_passes=False),
        out_type=[
            jax.ShapeDtypeStruct((2, NP, OUT), jnp.float32),
            jax.ShapeDtypeStruct((2, DREG, OUT), jnp.float32),
        ],
        scratch_types=[
            pltpu.VMEM((C2, OUT), jnp.float32),
            pltpu.VMEM((C2, OUT), jnp.float32),
            pltpu.VMEM((C2, OUT), jnp.float32),
            pltpu.VMEM((C2, OUT), jnp.float32),
            pltpu.VMEM((H, C), jnp.float32),
            pltpu.VMEM((NSUBC, C2), jnp.int32),
            pltpu.VMEM((NSUBC, C2), jnp.int32),
            pltpu.VMEM((NSUBC, C2), jnp.int32),
            pltpu.VMEM((16,), jnp.float32),
            pltpu.VMEM_SHARED((TROWS, OUT), jnp.float32),
            pltpu.SemaphoreType.DMA,
            pltpu.SemaphoreType.DMA,
        ],
    )
    def k2(vt_hbm, a_hbm, idx_hbm, dst_hbm, dden_hbm, gmax_hbm, num_hbm, den_hbm,
           vb0, vb1, mb, db, ab, idx_v, dst_v, dden_v, gv, t_sh, sem, sem2):
        cid = lax.axis_index("c")
        sid = lax.axis_index("s")
        wid = cid * 16 + sid
        iota = lax.iota(jnp.int32, 16)
        zero16 = jnp.zeros((16,), jnp.float32)
        zero16i = jnp.zeros((16,), jnp.int32)

        def zmb(i, _):
            r = i // 8
            cidx = (i % 8) * 16
            mb[r, pl.ds(cidx, 16)] = zero16
            db[r, pl.ds(cidx, 16)] = zero16
            return 0
        lax.fori_loop(0, C2 * 8, zmb, 0)

        base = sid * ROWS_T
        for p in range(NZP):
            pltpu.sync_copy(mb, t_sh.at[pl.ds(base + p * C2, C2)])
        dbase = NP + sid * DROWS_T
        pltpu.sync_copy(mb.at[pl.ds(0, DROWS_T)], t_sh.at[pl.ds(dbase, DROWS_T)])
        plsc.subcore_barrier()

        pltpu.sync_copy(gmax_hbm, gv)
        g16 = gv[...]

        def chunk_body(io, _):
            pltpu.sync_copy(a_hbm.at[wid, io], ab)
            r0 = wid * (NCHUNK * NSUBC) + io * NSUBC
            pltpu.sync_copy(idx_hbm.at[pl.ds(r0, NSUBC)], idx_v)
            pltpu.sync_copy(dst_hbm.at[pl.ds(r0, NSUBC)], dst_v)
            pltpu.sync_copy(dden_hbm.at[pl.ds(r0, NSUBC)], dden_v)

            vbs = [vb0, vb1]
            sems = [sem, sem2]
            pend = pltpu.async_copy(vt_hbm.at[idx_v.at[0]], vb0, sem)
            for isb in range(NSUBC):
                pend.wait()
                if isb + 1 < NSUBC:
                    pend = pltpu.async_copy(vt_hbm.at[idx_v.at[isb + 1]],
                                            vbs[(isb + 1) % 2], sems[(isb + 1) % 2])
                vb = vbs[isb % 2]
                acol0 = isb * C2

                def grp_body(g, _, vb=vb, isb=isb, acol0=acol0):
                    rows = g * 16 + iota
                    dv = plsc.load_gather(dst_v, [jnp.full((16,), isb, jnp.int32), rows])
                    bcol = (dv & 15) * 8
                    ex = []
                    for h in range(H):
                        hv = jnp.full((16,), h, jnp.int32)
                        av = plsc.load_gather(ab, [hv, acol0 + rows])
                        e = jnp.exp(av - g16)
                        ex.append(e)
                        plsc.store_scatter(db, [rows, bcol + h], e)
                    for j in range(OUT):
                        jv = jnp.full((16,), j, jnp.int32)
                        vc = plsc.load_gather(vb, [rows, jv])
                        plsc.store_scatter(mb, [rows, jv], vc * ex[j // HD])
                    return 0

                lax.fori_loop(0, NGRP2, grp_body, 0)
                pltpu.sync_copy(mb, t_sh.at[dst_v.at[isb]], add=True)
                pltpu.sync_copy(db, t_sh.at[dden_v.at[isb]], add=True)

                def zgrp_body(g, _, isb=isb):
                    rows = g * 16 + iota
                    dv = plsc.load_gather(dst_v, [jnp.full((16,), isb, jnp.int32), rows])
                    bcol = (dv & 15) * 8
                    for h in range(H):
                        plsc.store_scatter(db, [rows, bcol + h], zero16)
                    return 0

                lax.fori_loop(0, NGRP2, zgrp_body, 0)
            return 0

        lax.fori_loop(0, NCHUNK, chunk_body, 0)
        plsc.subcore_barrier()

        for p in range(NZP):
            st = base + p * C2
            pltpu.sync_copy(t_sh.at[pl.ds(st, C2)], mb)
            pltpu.sync_copy(mb, num_hbm.at[cid].at[pl.ds(st, C2)])
        pltpu.sync_copy(t_sh.at[pl.ds(dbase, DROWS_T)], mb.at[pl.ds(0, DROWS_T)])
        pltpu.sync_copy(mb.at[pl.ds(0, DROWS_T)],
                        den_hbm.at[cid].at[pl.ds(sid * DROWS_T, DROWS_T)])

    return k2(vt, a, idx2d, dst2d, dden2d, gmax16)


def _stage_c_body(is_final, num0_ref, num1_ref, den0_ref, den1_ref, x_ref,
                  nt_ref, wa_ref, sk_ref, ln_ref, o0_ref, agg_ref, o_ref):
    den = den0_ref[0] + den1_ref[0]
    deninv = 1.0 / jnp.maximum(den, 1e-30)
    hexp = (lax.broadcasted_iota(jnp.int32, (H, OUT), 1) // HD ==
            lax.broadcasted_iota(jnp.int32, (H, OUT), 0)).astype(jnp.float32)
    denfull = jnp.dot(deninv, hexp, preferred_element_type=jnp.float32)
    hmsg = (num0_ref[0] + num1_ref[0]) * denfull
    nt = nt_ref[...]
    xb = x_ref[...]
    hout = jnp.zeros_like(xb)
    alpha = jnp.zeros_like(nt)
    for t in range(NT):
        m = (nt == float(t)).astype(jnp.float32)
        hout = hout + m * jnp.dot(hmsg, wa_ref[t], preferred_element_type=jnp.float32)
        alpha = alpha + m * sk_ref[0, t]
    y = xb + hout * alpha + xb * (1.0 - alpha)
    mu = jnp.mean(y, axis=-1, keepdims=True)
    var = jnp.mean((y - mu) ** 2, axis=-1, keepdims=True)
    o = (y - mu) / jnp.sqrt(var + 1e-5) * ln_ref[0, :] + ln_ref[1, :]
    if is_final:
        mixed = agg_ref[0, 0] * o0_ref[...] + agg_ref[0, 1] * o
        mu2 = jnp.mean(mixed, axis=-1, keepdims=True)
        var2 = jnp.mean((mixed - mu2) ** 2, axis=-1, keepdims=True)
        o = (mixed - mu2) / jnp.sqrt(var2 + 1e-5) * ln_ref[2, :] + ln_ref[3, :]
    o_ref[...] = o


def _stage_c(is_final, num, den, x, nt2d, Wa, skpad, lnstack, o0, aggpad):
    B = 1000
    g = N // B
    return pl.pallas_call(
        functools.partial(_stage_c_body, is_final),
        grid=(g,),
        in_specs=[
            pl.BlockSpec((1, B, OUT), lambda i: (0, i, 0)),
            pl.BlockSpec((1, B, OUT), lambda i: (1, i, 0)),
            pl.BlockSpec((1, B, H), lambda i: (0, i, 0)),
            pl.BlockSpec((1, B, H), lambda i: (1, i, 0)),
            pl.BlockSpec((B, D), lambda i: (i, 0)),
            pl.BlockSpec((B, 1), lambda i: (i, 0)),
            pl.BlockSpec((NT, OUT, OUT), lambda i: (0, 0, 0)),
            pl.BlockSpec((1, NT), lambda i: (0, 0)),
            pl.BlockSpec((4, OUT), lambda i: (0, 0)),
            pl.BlockSpec((B, OUT), lambda i: (i, 0)),
            pl.BlockSpec((1, 2), lambda i: (0, 0)),
        ],
        out_specs=pl.BlockSpec((B, OUT), lambda i: (i, 0)),
        out_shape=jax.ShapeDtypeStruct((N, OUT), jnp.float32),
    )(num, num, den, den, x, nt2d, Wa, skpad, lnstack, o0, aggpad)


def _big_block_diag(Wper, scale):
    t = Wper * scale[:, :, None, None]
    eye = jnp.eye(H, dtype=t.dtype)
    big = jnp.einsum('heij,hg->hiegj', t, eye)
    return big.reshape(H * HD, ET * H * HD)


def _hgt_layer_sc(x, nt2d, idx2d, dst2d, Wk, Wq, Wv, Wa, Watt, Wmsg, pri,
                  skip, lng, lnb, is_final, o0, aggpad, agg_g, agg_b):
    sqrt_d = float(HD) ** 0.5
    BigA = _big_block_diag(Watt, pri / sqrt_d)
    BigM = _big_block_diag(Wmsg, jnp.ones_like(pri))
    kt, vt, q = _stage_a(x, nt2d, Wk, Wq, Wv, BigA, BigM)
    kt = kt.reshape(N * ET, OUT)
    vt = vt.reshape(N * ET, OUT)
    a, tmax = _sc_pass1(kt, q, idx2d[0], dst2d[0])
    gmax16 = jnp.broadcast_to(jnp.max(tmax), (16,))
    num, denraw = _sc_pass2(vt, a, idx2d[1], dst2d[1], dst2d[2], gmax16)
    den = denraw.reshape(2, DREG, 16, H).reshape(2, NP, H)
    skpad = jnp.concatenate([jax.nn.sigmoid(skip)[None, :]], axis=0)
    if is_final:
        lnstack = jnp.stack([lng, lnb, agg_g, agg_b], axis=0)
    else:
        lnstack = jnp.stack([lng, lnb, jnp.zeros_like(lng), jnp.zeros_like(lnb)], axis=0)
    return _stage_c(is_final, num, den, x, nt2d, Wa, skpad, lnstack, o0, aggpad)


def kernel(x, edge_index, ntype, etype, Wk0, Wq0, Wv0, Wa0, Watt0, Wmsg0, pri0, skip0, lng0, lnb0, Wk1, Wq1, Wv1, Wa1, Watt1, Wmsg1, pri1, skip1, lng1, lnb1, agg_w, agg_g, agg_b):
    src = edge_index[0]
    dst = edge_index[1]
    idx = src * ET + etype
    pad = EPAD - E
    idxp = jnp.concatenate([idx, jnp.zeros((pad,), jnp.int32)])
    dstp = jnp.concatenate([dst, jnp.full((pad,), N, jnp.int32)])
    ddenp = NP + (dstp >> 4)
    idx2d = (idxp.reshape(EPAD // SUB, SUB), idxp.reshape(EPAD // C2, C2))
    dst2d = (dstp.reshape(EPAD // SUB, SUB), dstp.reshape(EPAD // C2, C2),
             ddenp.reshape(EPAD // C2, C2))
    nt2d = ntype.astype(jnp.float32).reshape(N, 1)
    w = jax.nn.softmax(agg_w)
    aggpad = w.reshape(1, 2)
    zed = jnp.zeros((N, OUT), jnp.float32)
    o0 = _hgt_layer_sc(x, nt2d, idx2d, dst2d, Wk0, Wq0, Wv0, Wa0, Watt0,
                       Wmsg0, pri0, skip0, lng0, lnb0, False, zed, aggpad,
                       agg_g, agg_b)
    out = _hgt_layer_sc(o0, nt2d, idx2d, dst2d, Wk1, Wq1, Wv1, Wa1, Watt1,
                        Wmsg1, pri1, skip1, lng1, lnb1, True, o0, aggpad,
                        agg_g, agg_b)
    return out


# speedup vs baseline: 35.2083x; 2.2877x over previous
import functools

import jax
import jax.numpy as jnp
from jax import lax
from jax.experimental import pallas as pl
from jax.experimental.pallas import tpu as pltpu
from jax.experimental.pallas import tpu_sc as plsc

N = 10000
E = 320000
D = 128
OUT = 128
H = 8
HD = 16
NT = 3
ET = 4

NTILES = 32
EPT = 10240
EPAD = EPT * NTILES
C = 256
NCHUNK = EPT // C
NGRP = C // 16
SUB = 128
NSUB = C // SUB
C2 = 64
NSUBC = C // C2
NGRP2 = C2 // 16
NP = 10240
ROWS_T = NP // 16
NZP = ROWS_T // C2
DREG = NP // 16
DROWS_T = DREG // 16
TROWS = NP + DREG


def _stage_a_body(x_ref, nt_ref, wk_ref, wq_ref, wv_ref, biga_ref, bigm_ref,
                  kt_ref, vt_ref, q_ref):
    xb = x_ref[...]
    nt = nt_ref[...]
    k = jnp.zeros_like(xb)
    q = jnp.zeros_like(xb)
    v = jnp.zeros_like(xb)
    for t in range(NT):
        m = (nt == float(t)).astype(jnp.float32)
        k = k + m * jnp.dot(xb, wk_ref[t], preferred_element_type=jnp.float32)
        q = q + m * jnp.dot(xb, wq_ref[t], preferred_element_type=jnp.float32)
        v = v + m * jnp.dot(xb, wv_ref[t], preferred_element_type=jnp.float32)
    kt_ref[...] = jnp.dot(k, biga_ref[...], preferred_element_type=jnp.float32)
    vt_ref[...] = jnp.dot(v, bigm_ref[...], preferred_element_type=jnp.float32)
    q_ref[...] = q


def _stage_a(x, nt2d, Wk, Wq, Wv, BigA, BigM):
    B = 1000
    g = N // B
    return pl.pallas_call(
        _stage_a_body,
        grid=(g,),
        in_specs=[
            pl.BlockSpec((B, D), lambda i: (i, 0)),
            pl.BlockSpec((B, 1), lambda i: (i, 0)),
            pl.BlockSpec((NT, D, OUT), lambda i: (0, 0, 0)),
            pl.BlockSpec((NT, D, OUT), lambda i: (0, 0, 0)),
            pl.BlockSpec((NT, D, OUT), lambda i: (0, 0, 0)),
            pl.BlockSpec((OUT, ET * OUT), lambda i: (0, 0)),
            pl.BlockSpec((OUT, ET * OUT), lambda i: (0, 0)),
        ],
        out_specs=[
            pl.BlockSpec((B, ET * OUT), lambda i: (i, 0)),
            pl.BlockSpec((B, ET * OUT), lambda i: (i, 0)),
            pl.BlockSpec((B, OUT), lambda i: (i, 0)),
        ],
        out_shape=[
            jax.ShapeDtypeStruct((N, ET * OUT), jnp.float32),
            jax.ShapeDtypeStruct((N, ET * OUT), jnp.float32),
            jax.ShapeDtypeStruct((N, OUT), jnp.float32),
        ],
    )(x, nt2d, Wk, Wq, Wv, BigA, BigM)


def _sc_pass1(kt, q, idx2d, dst2d):
    mesh = plsc.VectorSubcoreMesh(core_axis_name="c", subcore_axis_name="s")

    @functools.partial(
        pl.kernel,
        mesh=mesh,
        compiler_params=pltpu.CompilerParams(needs_layout_passes=False),
        out_type=[
            jax.ShapeDtypeStruct((NTILES, NCHUNK, H, C), jnp.float32),
            jax.ShapeDtypeStruct((NTILES, 16), jnp.float32),
        ],
        scratch_types=[
            pltpu.VMEM((C, OUT), jnp.float32),
            pltpu.VMEM((C, OUT), jnp.float32),
            pltpu.VMEM((H, C), jnp.float32),
            pltpu.VMEM((EPT // SUB, SUB), jnp.int32),
            pltpu.VMEM((EPT // SUB, SUB), jnp.int32),
            pltpu.VMEM((16,), jnp.float32),
            pltpu.SemaphoreType.DMA,
        ],
    )
    def k1(kt_hbm, q_hbm, idx_hbm, dst_hbm, a_hbm, tmax_hbm,
           kb, qb, ab, idx_v, dst_v, tm_v, sem):
        cid = lax.axis_index("c")
        sid = lax.axis_index("s")
        wid = cid * 16 + sid
        iota = lax.iota(jnp.int32, 16)

        pltpu.sync_copy(idx_hbm.at[pl.ds(wid * (EPT // SUB), EPT // SUB)], idx_v)
        pltpu.sync_copy(dst_hbm.at[pl.ds(wid * (EPT // SUB), EPT // SUB)], dst_v)

        def chunk_body(i, tmaxes):
            ds = []
            for p in range(NSUB):
                ds.append(pltpu.async_copy(kt_hbm.at[idx_v.at[i * NSUB + p]],
                                           kb.at[pl.ds(p * SUB, SUB)], sem))
                ds.append(pltpu.async_copy(q_hbm.at[dst_v.at[i * NSUB + p]],
                                           qb.at[pl.ds(p * SUB, SUB)], sem))
            for d in ds:
                d.wait()

            def grp_body(g, tm):
                rows = g * 16 + iota
                acc = [jnp.zeros((16,), jnp.float32) for _ in range(H)]
                for j in range(OUT):
                    jv = ((iota + j) & (HD - 1)) + (j // HD) * HD
                    kc = plsc.load_gather(kb, [rows, jv])
                    qc = plsc.load_gather(qb, [rows, jv])
                    acc[j // HD] = acc[j // HD] + kc * qc
                new_tm = []
                for h in range(H):
                    hv = jnp.full((16,), h, jnp.int32)
                    plsc.store_scatter(ab, [hv, rows], acc[h])
                    new_tm.append(jnp.maximum(tm[h], acc[h]))
                return tuple(new_tm)

            tmaxes = lax.fori_loop(0, NGRP, grp_body, tmaxes)
            pltpu.sync_copy(ab, a_hbm.at[wid, i])
            return tmaxes

        init = tuple(jnp.full((16,), -jnp.inf, jnp.float32) for _ in range(H))
        tmaxes = lax.fori_loop(0, NCHUNK, chunk_body, init)
        tm = tmaxes[0]
        for h in range(1, H):
            tm = jnp.maximum(tm, tmaxes[h])
        tm_v[...] = tm
        pltpu.sync_copy(tm_v, tmax_hbm.at[wid])

    return k1(kt, q, idx2d, dst2d)


def _sc_pass2(vt, a, idx2d, dst2d, dden2d, gmax16):
    mesh = plsc.VectorSubcoreMesh(core_axis_name="c", subcore_axis_name="s")

    @functools.partial(
        pl.kernel,
        mesh=mesh,
        compiler_params=pltpu.CompilerParams(needs_layout_passes=False),
        out_type=[
            jax.ShapeDtypeStruct((2, NP, OUT), jnp.float32),
            jax.ShapeDtypeStruct((2, DREG, OUT), jnp.float32),
        ],
        scratch_types=[
            pltpu.VMEM((C2, OUT), jnp.float32),
            pltpu.VMEM((C2, OUT), jnp.float32),
            pltpu.VMEM((C2, OUT), jnp.float32),
            pltpu.VMEM((C2, OUT), jnp.float32),
            pltpu.VMEM((H, C), jnp.float32),
            pltpu.VMEM((NSUBC, C2), jnp.int32),
            pltpu.VMEM((NSUBC, C2), jnp.int32),
            pltpu.VMEM((NSUBC, C2), jnp.int32),
            pltpu.VMEM((16,), jnp.float32),
            pltpu.VMEM_SHARED((TROWS, OUT), jnp.float32),
            pltpu.SemaphoreType.DMA,
            pltpu.SemaphoreType.DMA,
        ],
    )
    def k2(vt_hbm, a_hbm, idx_hbm, dst_hbm, dden_hbm, gmax_hbm, num_hbm, den_hbm,
           vb0, vb1, mb, db, ab, idx_v, dst_v, dden_v, gv, t_sh, sem, sem2):
        cid = lax.axis_index("c")
        sid = lax.axis_index("s")
        wid = cid * 16 + sid
        iota = lax.iota(jnp.int32, 16)
        zero16 = jnp.zeros((16,), jnp.float32)
        zero16i = jnp.zeros((16,), jnp.int32)

        def zmb(i, _):
            r = i // 8
            cidx = (i % 8) * 16
            mb[r, pl.ds(cidx, 16)] = zero16
            db[r, pl.ds(cidx, 16)] = zero16
            return 0
        lax.fori_loop(0, C2 * 8, zmb, 0)

        base = sid * ROWS_T
        for p in range(NZP):
            pltpu.sync_copy(mb, t_sh.at[pl.ds(base + p * C2, C2)])
        dbase = NP + sid * DROWS_T
        pltpu.sync_copy(mb.at[pl.ds(0, DROWS_T)], t_sh.at[pl.ds(dbase, DROWS_T)])
        plsc.subcore_barrier()

        pltpu.sync_copy(gmax_hbm, gv)
        g16 = gv[...]

        def chunk_body(io, _):
            pltpu.sync_copy(a_hbm.at[wid, io], ab)
            r0 = wid * (NCHUNK * NSUBC) + io * NSUBC
            pltpu.sync_copy(idx_hbm.at[pl.ds(r0, NSUBC)], idx_v)
            pltpu.sync_copy(dst_hbm.at[pl.ds(r0, NSUBC)], dst_v)
            pltpu.sync_copy(dden_hbm.at[pl.ds(r0, NSUBC)], dden_v)

            vbs = [vb0, vb1]
            sems = [sem, sem2]
            pend = pltpu.async_copy(vt_hbm.at[idx_v.at[0]], vb0, sem)
            for isb in range(NSUBC):
                pend.wait()
                if isb + 1 < NSUBC:
                    pend = pltpu.async_copy(vt_hbm.at[idx_v.at[isb + 1]],
                                            vbs[(isb + 1) % 2], sems[(isb + 1) % 2])
                vb = vbs[isb % 2]
                acol0 = isb * C2

                def grp_body(g, _, vb=vb, isb=isb, acol0=acol0):
                    rows = g * 16 + iota
                    dv = plsc.load_gather(dst_v, [jnp.full((16,), isb, jnp.int32), rows])
                    bcol = (dv & 15) * 8
                    ex = []
                    for h in range(H):
                        hv = jnp.full((16,), h, jnp.int32)
                        av = plsc.load_gather(ab, [hv, acol0 + rows])
                        e = jnp.exp(av - g16)
                        ex.append(e)
                        plsc.store_scatter(db, [rows, bcol + h], e)
                    for j in range(OUT):
                        jv = ((iota + j) & (HD - 1)) + (j // HD) * HD
                        vc = plsc.load_gather(vb, [rows, jv])
                        plsc.store_scatter(mb, [rows, jv], vc * ex[j // HD])
                    return 0

                lax.fori_loop(0, NGRP2, grp_body, 0)
                pltpu.sync_copy(mb, t_sh.at[dst_v.at[isb]], add=True)
                pltpu.sync_copy(db, t_sh.at[dden_v.at[isb]], add=True)

                def zgrp_body(g, _, isb=isb):
                    rows = g * 16 + iota
                    dv = plsc.load_gather(dst_v, [jnp.full((16,), isb, jnp.int32), rows])
                    bcol = (dv & 15) * 8
                    for h in range(H):
                        plsc.store_scatter(db, [rows, bcol + h], zero16)
                    return 0

                lax.fori_loop(0, NGRP2, zgrp_body, 0)
            return 0

        lax.fori_loop(0, NCHUNK, chunk_body, 0)
        plsc.subcore_barrier()

        for p in range(NZP):
            st = base + p * C2
            pltpu.sync_copy(t_sh.at[pl.ds(st, C2)], mb)
            pltpu.sync_copy(mb, num_hbm.at[cid].at[pl.ds(st, C2)])
        pltpu.sync_copy(t_sh.at[pl.ds(dbase, DROWS_T)], mb.at[pl.ds(0, DROWS_T)])
        pltpu.sync_copy(mb.at[pl.ds(0, DROWS_T)],
                        den_hbm.at[cid].at[pl.ds(sid * DROWS_T, DROWS_T)])

    return k2(vt, a, idx2d, dst2d, dden2d, gmax16)


def _stage_c_body(is_final, num0_ref, num1_ref, den0_ref, den1_ref, x_ref,
                  nt_ref, wa_ref, sk_ref, ln_ref, o0_ref, agg_ref, o_ref):
    den = den0_ref[0] + den1_ref[0]
    deninv = 1.0 / jnp.maximum(den, 1e-30)
    hexp = (lax.broadcasted_iota(jnp.int32, (H, OUT), 1) // HD ==
            lax.broadcasted_iota(jnp.int32, (H, OUT), 0)).astype(jnp.float32)
    denfull = jnp.dot(deninv, hexp, preferred_element_type=jnp.float32)
    hmsg = (num0_ref[0] + num1_ref[0]) * denfull
    nt = nt_ref[...]
    xb = x_ref[...]
    hout = jnp.zeros_like(xb)
    alpha = jnp.zeros_like(nt)
    for t in range(NT):
        m = (nt == float(t)).astype(jnp.float32)
        hout = hout + m * jnp.dot(hmsg, wa_ref[t], preferred_element_type=jnp.float32)
        alpha = alpha + m * sk_ref[0, t]
    y = xb + hout * alpha + xb * (1.0 - alpha)
    mu = jnp.mean(y, axis=-1, keepdims=True)
    var = jnp.mean((y - mu) ** 2, axis=-1, keepdims=True)
    o = (y - mu) / jnp.sqrt(var + 1e-5) * ln_ref[0, :] + ln_ref[1, :]
    if is_final:
        mixed = agg_ref[0, 0] * o0_ref[...] + agg_ref[0, 1] * o
        mu2 = jnp.mean(mixed, axis=-1, keepdims=True)
        var2 = jnp.mean((mixed - mu2) ** 2, axis=-1, keepdims=True)
        o = (mixed - mu2) / jnp.sqrt(var2 + 1e-5) * ln_ref[2, :] + ln_ref[3, :]
    o_ref[...] = o


def _stage_c(is_final, num, den, x, nt2d, Wa, skpad, lnstack, o0, aggpad):
    B = 1000
    g = N // B
    return pl.pallas_call(
        functools.partial(_stage_c_body, is_final),
        grid=(g,),
        in_specs=[
            pl.BlockSpec((1, B, OUT), lambda i: (0, i, 0)),
            pl.BlockSpec((1, B, OUT), lambda i: (1, i, 0)),
            pl.BlockSpec((1, B, H), lambda i: (0, i, 0)),
            pl.BlockSpec((1, B, H), lambda i: (1, i, 0)),
            pl.BlockSpec((B, D), lambda i: (i, 0)),
            pl.BlockSpec((B, 1), lambda i: (i, 0)),
            pl.BlockSpec((NT, OUT, OUT), lambda i: (0, 0, 0)),
            pl.BlockSpec((1, NT), lambda i: (0, 0)),
            pl.BlockSpec((4, OUT), lambda i: (0, 0)),
            pl.BlockSpec((B, OUT), lambda i: (i, 0)),
            pl.BlockSpec((1, 2), lambda i: (0, 0)),
        ],
        out_specs=pl.BlockSpec((B, OUT), lambda i: (i, 0)),
        out_shape=jax.ShapeDtypeStruct((N, OUT), jnp.float32),
    )(num, num, den, den, x, nt2d, Wa, skpad, lnstack, o0, aggpad)


def _big_block_diag(Wper, scale):
    t = Wper * scale[:, :, None, None]
    eye = jnp.eye(H, dtype=t.dtype)
    big = jnp.einsum('heij,hg->hiegj', t, eye)
    return big.reshape(H * HD, ET * H * HD)


def _hgt_layer_sc(x, nt2d, idx2d, dst2d, Wk, Wq, Wv, Wa, Watt, Wmsg, pri,
                  skip, lng, lnb, is_final, o0, aggpad, agg_g, agg_b):
    sqrt_d = float(HD) ** 0.5
    BigA = _big_block_diag(Watt, pri / sqrt_d)
    BigM = _big_block_diag(Wmsg, jnp.ones_like(pri))
    kt, vt, q = _stage_a(x, nt2d, Wk, Wq, Wv, BigA, BigM)
    kt = kt.reshape(N * ET, OUT)
    vt = vt.reshape(N * ET, OUT)
    a, tmax = _sc_pass1(kt, q, idx2d[0], dst2d[0])
    gmax16 = jnp.broadcast_to(jnp.max(tmax), (16,))
    num, denraw = _sc_pass2(vt, a, idx2d[1], dst2d[1], dst2d[2], gmax16)
    den = denraw.reshape(2, DREG, 16, H).reshape(2, NP, H)
    skpad = jnp.concatenate([jax.nn.sigmoid(skip)[None, :]], axis=0)
    if is_final:
        lnstack = jnp.stack([lng, lnb, agg_g, agg_b], axis=0)
    else:
        lnstack = jnp.stack([lng, lnb, jnp.zeros_like(lng), jnp.zeros_like(lnb)], axis=0)
    return _stage_c(is_final, num, den, x, nt2d, Wa, skpad, lnstack, o0, aggpad)


def kernel(x, edge_index, ntype, etype, Wk0, Wq0, Wv0, Wa0, Watt0, Wmsg0, pri0, skip0, lng0, lnb0, Wk1, Wq1, Wv1, Wa1, Watt1, Wmsg1, pri1, skip1, lng1, lnb1, agg_w, agg_g, agg_b):
    src = edge_index[0]
    dst = edge_index[1]
    idx = src * ET + etype
    pad = EPAD - E
    idxp = jnp.concatenate([idx, jnp.zeros((pad,), jnp.int32)])
    dstp = jnp.concatenate([dst, jnp.full((pad,), N, jnp.int32)])
    ddenp = NP + (dstp >> 4)
    idx2d = (idxp.reshape(EPAD // SUB, SUB), idxp.reshape(EPAD // C2, C2))
    dst2d = (dstp.reshape(EPAD // SUB, SUB), dstp.reshape(EPAD // C2, C2),
             ddenp.reshape(EPAD // C2, C2))
    nt2d = ntype.astype(jnp.float32).reshape(N, 1)
    w = jax.nn.softmax(agg_w)
    aggpad = w.reshape(1, 2)
    zed = jnp.zeros((N, OUT), jnp.float32)
    o0 = _hgt_layer_sc(x, nt2d, idx2d, dst2d, Wk0, Wq0, Wv0, Wa0, Watt0,
                       Wmsg0, pri0, skip0, lng0, lnb0, False, zed, aggpad,
                       agg_g, agg_b)
    out = _hgt_layer_sc(o0, nt2d, idx2d, dst2d, Wk1, Wq1, Wv1, Wa1, Watt1,
                        Wmsg1, pri1, skip1, lng1, lnb1, True, o0, aggpad,
                        agg_g, agg_b)
    return out
